# Initial kernel scaffold; baseline (speedup 1.0000x reference)
import jax
import jax.numpy as jnp
from jax.experimental import pallas as pl


def kernel(x):
    raise NotImplementedError("write your pallas kernel here")


# speedup vs baseline: 2.0751x; 2.0751x over previous
import jax
import jax.numpy as jnp
from jax.experimental import pallas as pl
from jax.experimental.pallas import tpu as pltpu


def _maxpool_kernel(x_ref, o_ref):
    k, g, _, w = x_ref.shape
    lane = jax.lax.broadcasted_iota(jnp.int32, (k, g, 128), 2)
    idx0 = (2 * lane) & 127
    idx1 = (2 * lane + 32) & 127
    for u in range(4):
        e = x_ref[:, :, 2 * u, :]
        o = x_ref[:, :, 2 * u + 1, :]
        a = jnp.maximum(e, o)
        m = jnp.maximum(a, pltpu.roll(a, w - 1, 2))
        t0 = m[:, :, 0:128]
        t1 = m[:, :, 96:224]
        g0 = jnp.take_along_axis(t0, idx0, axis=2)
        g1 = jnp.take_along_axis(t1, idx1, axis=2)
        out = jnp.where(lane < 64, g0, g1)
        o_ref[:, :, u, :] = out[:, :, 0:112]


def kernel(x):
    N, C, H, W = x.shape
    HO, WO = H // 2, W // 2
    NC = N * C
    K = 8
    xv = x.reshape(NC, H // 8, 8, W)
    out = pl.pallas_call(
        _maxpool_kernel,
        grid=(NC // K,),
        in_specs=[pl.BlockSpec((K, H // 8, 8, W), lambda i: (i, 0, 0, 0))],
        out_specs=pl.BlockSpec((K, H // 8, 4, WO), lambda i: (i, 0, 0, 0)),
        out_shape=jax.ShapeDtypeStruct((NC, H // 8, 4, WO), x.dtype),
        compiler_params=pltpu.CompilerParams(
            dimension_semantics=("parallel",),
        ),
    )(xv)
    return out.reshape(N, C, HO, WO)
